# baseline (speedup 1.0000x reference)
import functools
import math

import jax
import jax.numpy as jnp
import numpy as np
from jax import lax
from jax.experimental import pallas as pl
from jax.experimental.pallas import tpu as pltpu
from jax.experimental.pallas import tpu_sc as plsc

_B, _H, _S, _D = 4, 16, 2048, 128
_NROWS = _B * _H * _S

_CENTROIDS = np.array(
    [-2.7326, -2.069, -1.618, -1.2562, -0.9423, -0.6568, -0.3881, -0.1284,
     0.1284, 0.3881, 0.6568, 0.9423, 1.2562, 1.618, 2.069, 2.7326],
    dtype=np.float32)
_BOUNDS = ((_CENTROIDS[:-1] + _CENTROIDS[1:]) / 2).astype(np.float32)
_PB = _BOUNDS[8:]
_C8 = float(_CENTROIDS[8])
_DCP = [float(x) for x in (_CENTROIDS[9:] - _CENTROIDS[8:15])]
_PB2D = [float(x) for x in (_PB.astype(np.float64) ** 2 / _D)]
_CPOS = [float(x) for x in _CENTROIDS[8:]]
_INV_SQRT_D = float(np.float32(1.0 / math.sqrt(_D)))

_NW = 32
_CHUNK = 128
_CS = _CHUNK * _D


def _rsqrt_newton(ssc):
    ii = lax.bitcast_convert_type(ssc, jnp.int32)
    ii = 0x5F3759DF - lax.shift_right_logical(ii, 1)
    y = lax.bitcast_convert_type(ii, jnp.float32)
    for _ in range(3):
        y = y * (1.5 - 0.5 * ssc * y * y)
    return y


def _sc_compute_chunk(inb_b, outb_b):
    def row_body(r, carry):
        base = r * _D
        xs = [inb_b[pl.ds(base + 16 * i, 16)] for i in range(8)]
        sm = ((xs[0] + xs[1]) + (xs[2] + xs[3])) + (
            (xs[4] + xs[5]) + (xs[6] + xs[7]))
        sq = [x * x for x in xs]
        sqm = ((sq[0] + sq[1]) + (sq[2] + sq[3])) + (
            (sq[4] + sq[5]) + (sq[6] + sq[7]))
        tot = jnp.sum(sm)
        tot2 = jnp.sum(sqm)
        mean = tot * (1.0 / _D)
        ssc = jnp.maximum(tot2 - mean * tot, 1e-30)
        mag = jnp.maximum(ssc * _rsqrt_newton(ssc), 1e-8)
        rm = mag * _INV_SQRT_D
        tb = [pb2d * ssc for pb2d in _PB2D]
        cv = [c * rm for c in _CPOS]
        for i in range(8):
            xc = xs[i] - mean
            t = xc * xc
            m1 = t > tb[3]
            m2 = t > jnp.where(m1, tb[5], tb[1])
            m3 = t > jnp.where(m2, jnp.where(m1, tb[6], tb[2]),
                               jnp.where(m1, tb[4], tb[0]))
            s0 = jnp.where(m3, cv[1], cv[0])
            s1 = jnp.where(m3, cv[3], cv[2])
            s2 = jnp.where(m3, cv[5], cv[4])
            s3 = jnp.where(m3, cv[7], cv[6])
            vmag = jnp.where(m1, jnp.where(m2, s3, s2),
                             jnp.where(m2, s1, s0))
            val = jnp.sign(xc) * vmag + mean
            outb_b[pl.ds(base + 16 * i, 16)] = val
        return carry

    lax.fori_loop(0, _CHUNK, row_body, 0)


def _sc_make(n_rows):
    rpw = n_rows // _NW
    nch = rpw // _CHUNK
    assert rpw % _CHUNK == 0 and nch % 2 == 0
    mesh = plsc.VectorSubcoreMesh(core_axis_name="c", subcore_axis_name="s")
    out = jax.ShapeDtypeStruct((n_rows * _D,), jnp.float32)

    @functools.partial(
        pl.kernel, mesh=mesh,
        out_type=out,
        compiler_params=pltpu.CompilerParams(needs_layout_passes=False),
        scratch_types=[
            pltpu.VMEM((_CS,), jnp.float32),
            pltpu.VMEM((_CS,), jnp.float32),
            pltpu.VMEM((_CS,), jnp.float32),
            pltpu.VMEM((_CS,), jnp.float32),
            pltpu.SemaphoreType.DMA,
            pltpu.SemaphoreType.DMA,
            pltpu.SemaphoreType.DMA,
            pltpu.SemaphoreType.DMA,
        ])
    def sc_kernel(src, dst, inb0, inb1, outb0, outb1, is0, is1, os0, os1):
        cid = lax.axis_index("c")
        sid = lax.axis_index("s")
        wid = sid * 2 + cid
        base = wid * (rpw * _D)
        inbs = (inb0, inb1)
        outbs = (outb0, outb1)
        isems = (is0, is1)
        osems = (os0, os1)

        pltpu.async_copy(src.at[pl.ds(base, _CS)], inbs[0], isems[0])

        def pair_body(p, carry):
            for b in (0, 1):
                i = 2 * p + b
                nb = 1 - b
                nxt = jnp.minimum(i + 1, nch - 1)
                pltpu.async_copy(
                    src.at[pl.ds(base + nxt * _CS, _CS)],
                    inbs[nb], isems[nb])
                pltpu.make_async_copy(
                    src.at[pl.ds(base + i * _CS, _CS)],
                    inbs[b], isems[b]).wait()
                @pl.when(i >= 2)
                def _():
                    pltpu.make_async_copy(
                        outbs[b],
                        dst.at[pl.ds(base + (i - 2) * _CS, _CS)],
                        osems[b]).wait()
                _sc_compute_chunk(inbs[b], outbs[b])
                pltpu.async_copy(
                    outbs[b],
                    dst.at[pl.ds(base + i * _CS, _CS)], osems[b])
            return carry

        lax.fori_loop(0, nch // 2, pair_body, 0)
        pltpu.make_async_copy(
            src.at[pl.ds(base, _CS)], inbs[0], isems[0]).wait()
        pltpu.make_async_copy(
            outbs[0],
            dst.at[pl.ds(base + (nch - 2) * _CS, _CS)], osems[0]).wait()
        pltpu.make_async_copy(
            outbs[1],
            dst.at[pl.ds(base + (nch - 1) * _CS, _CS)], osems[1]).wait()

    return sc_kernel


def _quant_dequant(x):
    mean = jnp.mean(x, axis=-1, keepdims=True)
    xc = x - mean
    ss = jnp.sum(xc * xc, axis=-1, keepdims=True)
    mag = jnp.maximum(jnp.sqrt(ss), 1e-8)
    rm = mag * _INV_SQRT_D
    a = jnp.abs(xc)
    acc = jnp.broadcast_to(_C8 * rm, x.shape)
    for j in range(7):
        acc = acc + jnp.where(a > float(_PB[j]) * rm, _DCP[j] * rm, 0.0)
    return jnp.where(xc > 0, acc, -acc) + mean


def _tc_body(v_ref, vo_ref):
    vo_ref[...] = _quant_dequant(v_ref[...])


def _tc_run(v2d):
    n = v2d.shape[0]
    blk = 2048
    spec = pl.BlockSpec((blk, _D), lambda i: (i, 0))
    return pl.pallas_call(
        _tc_body,
        grid=(n // blk,),
        in_specs=[spec],
        out_specs=spec,
        out_shape=jax.ShapeDtypeStruct((n, _D), jnp.float32),
    )(v2d)


@jax.jit
def _run(k1d, v2d):
    ko = _sc_make(_NROWS)(k1d)
    vo = _tc_run(v2d)
    return ko, vo


def kernel(input_pos, k_val, v_val, k_packed, v_packed, k_mag, v_mag,
           k_mean, v_mean):
    shape = k_val.shape
    ko, vo = _run(k_val.reshape(-1), v_val.reshape(-1, _D))
    return ko.reshape(shape), vo.reshape(shape)


# speedup vs baseline: 1.0019x; 1.0019x over previous
import functools
import math

import jax
import jax.numpy as jnp
import numpy as np
from jax import lax
from jax.experimental import pallas as pl
from jax.experimental.pallas import tpu as pltpu
from jax.experimental.pallas import tpu_sc as plsc

_B, _H, _S, _D = 4, 16, 2048, 128
_NROWS = _B * _H * _S

_CENTROIDS = np.array(
    [-2.7326, -2.069, -1.618, -1.2562, -0.9423, -0.6568, -0.3881, -0.1284,
     0.1284, 0.3881, 0.6568, 0.9423, 1.2562, 1.618, 2.069, 2.7326],
    dtype=np.float32)
_BOUNDS = ((_CENTROIDS[:-1] + _CENTROIDS[1:]) / 2).astype(np.float32)
_PB = _BOUNDS[8:]
_C8 = float(_CENTROIDS[8])
_DCP = [float(x) for x in (_CENTROIDS[9:] - _CENTROIDS[8:15])]
_PB2D = [float(x) for x in (_PB.astype(np.float64) ** 2 / _D)]
_CPOS = [float(x) for x in _CENTROIDS[8:]]
_INV_SQRT_D = float(np.float32(1.0 / math.sqrt(_D)))

_NW = 32
_CHUNK = 128
_CS = _CHUNK * _D


def _rsqrt_newton(ssc):
    ii = lax.bitcast_convert_type(ssc, jnp.int32)
    ii = 0x5F3759DF - lax.shift_right_logical(ii, 1)
    y = lax.bitcast_convert_type(ii, jnp.float32)
    for _ in range(3):
        y = y * (1.5 - 0.5 * ssc * y * y)
    return y


def _sc_compute_chunk(inb_b, outb_b):
    def row_body(r, carry):
        base = r * _D
        xs = [inb_b[pl.ds(base + 16 * i, 16)] for i in range(8)]
        sm = ((xs[0] + xs[1]) + (xs[2] + xs[3])) + (
            (xs[4] + xs[5]) + (xs[6] + xs[7]))
        sq = [x * x for x in xs]
        sqm = ((sq[0] + sq[1]) + (sq[2] + sq[3])) + (
            (sq[4] + sq[5]) + (sq[6] + sq[7]))
        tot = jnp.sum(sm)
        tot2 = jnp.sum(sqm)
        mean = tot * (1.0 / _D)
        ssc = jnp.maximum(tot2 - mean * tot, 1e-30)
        mag = jnp.maximum(ssc * _rsqrt_newton(ssc), 1e-8)
        rm = mag * _INV_SQRT_D
        ones = jnp.full((16,), 1.0, jnp.float32)
        mean_v = ones * mean
        tb = [ones * (pb2d * ssc) for pb2d in _PB2D]
        cv = [ones * (c * rm) for c in _CPOS]
        for i in range(8):
            xc = xs[i] - mean_v
            t = xc * xc
            m1 = t > tb[3]
            m2 = t > jnp.where(m1, tb[5], tb[1])
            m3 = t > jnp.where(m2, jnp.where(m1, tb[6], tb[2]),
                               jnp.where(m1, tb[4], tb[0]))
            s0 = jnp.where(m3, cv[1], cv[0])
            s1 = jnp.where(m3, cv[3], cv[2])
            s2 = jnp.where(m3, cv[5], cv[4])
            s3 = jnp.where(m3, cv[7], cv[6])
            vmag = jnp.where(m1, jnp.where(m2, s3, s2),
                             jnp.where(m2, s1, s0))
            val = jnp.sign(xc) * vmag + mean_v
            outb_b[pl.ds(base + 16 * i, 16)] = val
        return carry

    lax.fori_loop(0, _CHUNK, row_body, 0)


def _sc_make(n_rows):
    rpw = n_rows // _NW
    nch = rpw // _CHUNK
    assert rpw % _CHUNK == 0 and nch % 2 == 0
    mesh = plsc.VectorSubcoreMesh(core_axis_name="c", subcore_axis_name="s")
    out = jax.ShapeDtypeStruct((n_rows * _D,), jnp.float32)

    @functools.partial(
        pl.kernel, mesh=mesh,
        out_type=out,
        compiler_params=pltpu.CompilerParams(needs_layout_passes=False),
        scratch_types=[
            pltpu.VMEM((_CS,), jnp.float32),
            pltpu.VMEM((_CS,), jnp.float32),
            pltpu.VMEM((_CS,), jnp.float32),
            pltpu.VMEM((_CS,), jnp.float32),
            pltpu.SemaphoreType.DMA,
            pltpu.SemaphoreType.DMA,
            pltpu.SemaphoreType.DMA,
            pltpu.SemaphoreType.DMA,
        ])
    def sc_kernel(src, dst, inb0, inb1, outb0, outb1, is0, is1, os0, os1):
        cid = lax.axis_index("c")
        sid = lax.axis_index("s")
        wid = sid * 2 + cid
        base = wid * (rpw * _D)
        inbs = (inb0, inb1)
        outbs = (outb0, outb1)
        isems = (is0, is1)
        osems = (os0, os1)

        pltpu.async_copy(src.at[pl.ds(base, _CS)], inbs[0], isems[0])

        def pair_body(p, carry):
            for b in (0, 1):
                i = 2 * p + b
                nb = 1 - b
                nxt = jnp.minimum(i + 1, nch - 1)
                pltpu.async_copy(
                    src.at[pl.ds(base + nxt * _CS, _CS)],
                    inbs[nb], isems[nb])
                pltpu.make_async_copy(
                    src.at[pl.ds(base + i * _CS, _CS)],
                    inbs[b], isems[b]).wait()
                @pl.when(i >= 2)
                def _():
                    pltpu.make_async_copy(
                        outbs[b],
                        dst.at[pl.ds(base + (i - 2) * _CS, _CS)],
                        osems[b]).wait()
                _sc_compute_chunk(inbs[b], outbs[b])
                pltpu.async_copy(
                    outbs[b],
                    dst.at[pl.ds(base + i * _CS, _CS)], osems[b])
            return carry

        lax.fori_loop(0, nch // 2, pair_body, 0)
        pltpu.make_async_copy(
            src.at[pl.ds(base, _CS)], inbs[0], isems[0]).wait()
        pltpu.make_async_copy(
            outbs[0],
            dst.at[pl.ds(base + (nch - 2) * _CS, _CS)], osems[0]).wait()
        pltpu.make_async_copy(
            outbs[1],
            dst.at[pl.ds(base + (nch - 1) * _CS, _CS)], osems[1]).wait()

    return sc_kernel


def _quant_dequant(x):
    mean = jnp.mean(x, axis=-1, keepdims=True)
    xc = x - mean
    ss = jnp.sum(xc * xc, axis=-1, keepdims=True)
    mag = jnp.maximum(jnp.sqrt(ss), 1e-8)
    rm = mag * _INV_SQRT_D
    a = jnp.abs(xc)
    acc = jnp.broadcast_to(_C8 * rm, x.shape)
    for j in range(7):
        acc = acc + jnp.where(a > float(_PB[j]) * rm, _DCP[j] * rm, 0.0)
    return jnp.where(xc > 0, acc, -acc) + mean


def _tc_body(v_ref, vo_ref):
    vo_ref[...] = _quant_dequant(v_ref[...])


def _tc_run(v2d):
    n = v2d.shape[0]
    blk = 2048
    spec = pl.BlockSpec((blk, _D), lambda i: (i, 0))
    return pl.pallas_call(
        _tc_body,
        grid=(n // blk,),
        in_specs=[spec],
        out_specs=spec,
        out_shape=jax.ShapeDtypeStruct((n, _D), jnp.float32),
    )(v2d)


@jax.jit
def _run(k1d, v2d):
    ko = _sc_make(_NROWS)(k1d)
    vo = _tc_run(v2d)
    return ko, vo


def kernel(input_pos, k_val, v_val, k_packed, v_packed, k_mag, v_mag,
           k_mean, v_mean):
    shape = k_val.shape
    ko, vo = _run(k_val.reshape(-1), v_val.reshape(-1, _D))
    return ko.reshape(shape), vo.reshape(shape)


# speedup vs baseline: 1.0593x; 1.0572x over previous
import functools
import math

import jax
import jax.numpy as jnp
import numpy as np
from jax import lax
from jax.experimental import pallas as pl
from jax.experimental.pallas import tpu as pltpu
from jax.experimental.pallas import tpu_sc as plsc

_B, _H, _S, _D = 4, 16, 2048, 128
_NROWS = _B * _H * _S

_CENTROIDS = np.array(
    [-2.7326, -2.069, -1.618, -1.2562, -0.9423, -0.6568, -0.3881, -0.1284,
     0.1284, 0.3881, 0.6568, 0.9423, 1.2562, 1.618, 2.069, 2.7326],
    dtype=np.float32)
_BOUNDS = ((_CENTROIDS[:-1] + _CENTROIDS[1:]) / 2).astype(np.float32)
_PB = _BOUNDS[8:]
_C8 = float(_CENTROIDS[8])
_DCP = [float(x) for x in (_CENTROIDS[9:] - _CENTROIDS[8:15])]
_PB2D = [float(x) for x in (_PB.astype(np.float64) ** 2 / _D)]
_CPOS = [float(x) for x in _CENTROIDS[8:]]
_INV_SQRT_D = float(np.float32(1.0 / math.sqrt(_D)))

_NW = 32
_CHUNK = 128
_CS = _CHUNK * _D


def _rsqrt_newton(ssc):
    ii = lax.bitcast_convert_type(ssc, jnp.int32)
    ii = 0x5F3759DF - lax.shift_right_logical(ii, 1)
    y = lax.bitcast_convert_type(ii, jnp.float32)
    for _ in range(3):
        y = y * (1.5 - 0.5 * ssc * y * y)
    return y


def _sc_compute_chunk(inb_b, outb_b):
    def row_body(r):
        base = r * _D
        xs = [inb_b[pl.ds(base + 16 * i, 16)] for i in range(8)]
        sm = ((xs[0] + xs[1]) + (xs[2] + xs[3])) + (
            (xs[4] + xs[5]) + (xs[6] + xs[7]))
        sq = [x * x for x in xs]
        sqm = ((sq[0] + sq[1]) + (sq[2] + sq[3])) + (
            (sq[4] + sq[5]) + (sq[6] + sq[7]))
        tot = jnp.sum(sm)
        tot2 = jnp.sum(sqm)
        mean = tot * (1.0 / _D)
        ssc = jnp.maximum(tot2 - mean * tot, 1e-30)
        mag = jnp.maximum(ssc * _rsqrt_newton(ssc), 1e-8)
        rm = mag * _INV_SQRT_D
        ones = jnp.full((16,), 1.0, jnp.float32)
        mean_v = ones * mean
        rm_v = ones * rm
        tbs = [ones * (pb2d * ssc) for pb2d in _PB2D]
        for i in range(8):
            xc = xs[i] - mean_v
            t = xc * xc
            acc = jnp.full((16,), _C8, jnp.float32)
            for j in range(7):
                acc = acc + jnp.where(t > tbs[j], _DCP[j], 0.0)
            val = jnp.where(xc > 0, acc, -acc) * rm_v + mean_v
            outb_b[pl.ds(base + 16 * i, 16)] = val

    plsc.parallel_loop(0, _CHUNK, 1, unroll=2)(row_body)


def _sc_make(n_rows):
    rpw = n_rows // _NW
    nch = rpw // _CHUNK
    assert rpw % _CHUNK == 0 and nch % 2 == 0
    mesh = plsc.VectorSubcoreMesh(core_axis_name="c", subcore_axis_name="s")
    out = jax.ShapeDtypeStruct((n_rows * _D,), jnp.float32)

    @functools.partial(
        pl.kernel, mesh=mesh,
        out_type=out,
        compiler_params=pltpu.CompilerParams(needs_layout_passes=False),
        scratch_types=[
            pltpu.VMEM((_CS,), jnp.float32),
            pltpu.VMEM((_CS,), jnp.float32),
            pltpu.VMEM((_CS,), jnp.float32),
            pltpu.VMEM((_CS,), jnp.float32),
            pltpu.SemaphoreType.DMA,
            pltpu.SemaphoreType.DMA,
            pltpu.SemaphoreType.DMA,
            pltpu.SemaphoreType.DMA,
        ])
    def sc_kernel(src, dst, inb0, inb1, outb0, outb1, is0, is1, os0, os1):
        cid = lax.axis_index("c")
        sid = lax.axis_index("s")
        wid = sid * 2 + cid
        base = wid * (rpw * _D)
        inbs = (inb0, inb1)
        outbs = (outb0, outb1)
        isems = (is0, is1)
        osems = (os0, os1)

        pltpu.async_copy(src.at[pl.ds(base, _CS)], inbs[0], isems[0])

        def pair_body(p, carry):
            for b in (0, 1):
                i = 2 * p + b
                nb = 1 - b
                nxt = jnp.minimum(i + 1, nch - 1)
                pltpu.async_copy(
                    src.at[pl.ds(base + nxt * _CS, _CS)],
                    inbs[nb], isems[nb])
                pltpu.make_async_copy(
                    src.at[pl.ds(base + i * _CS, _CS)],
                    inbs[b], isems[b]).wait()
                @pl.when(i >= 2)
                def _():
                    pltpu.make_async_copy(
                        outbs[b],
                        dst.at[pl.ds(base + (i - 2) * _CS, _CS)],
                        osems[b]).wait()
                _sc_compute_chunk(inbs[b], outbs[b])
                pltpu.async_copy(
                    outbs[b],
                    dst.at[pl.ds(base + i * _CS, _CS)], osems[b])
            return carry

        lax.fori_loop(0, nch // 2, pair_body, 0)
        pltpu.make_async_copy(
            src.at[pl.ds(base, _CS)], inbs[0], isems[0]).wait()
        pltpu.make_async_copy(
            outbs[0],
            dst.at[pl.ds(base + (nch - 2) * _CS, _CS)], osems[0]).wait()
        pltpu.make_async_copy(
            outbs[1],
            dst.at[pl.ds(base + (nch - 1) * _CS, _CS)], osems[1]).wait()

    return sc_kernel


def _quant_dequant(x):
    mean = jnp.mean(x, axis=-1, keepdims=True)
    xc = x - mean
    ss = jnp.sum(xc * xc, axis=-1, keepdims=True)
    mag = jnp.maximum(jnp.sqrt(ss), 1e-8)
    rm = mag * _INV_SQRT_D
    a = jnp.abs(xc)
    acc = jnp.broadcast_to(_C8 * rm, x.shape)
    for j in range(7):
        acc = acc + jnp.where(a > float(_PB[j]) * rm, _DCP[j] * rm, 0.0)
    return jnp.where(xc > 0, acc, -acc) + mean


def _tc_body(v_ref, vo_ref):
    vo_ref[...] = _quant_dequant(v_ref[...])


def _tc_run(v2d):
    n = v2d.shape[0]
    blk = 2048
    spec = pl.BlockSpec((blk, _D), lambda i: (i, 0))
    return pl.pallas_call(
        _tc_body,
        grid=(n // blk,),
        in_specs=[spec],
        out_specs=spec,
        out_shape=jax.ShapeDtypeStruct((n, _D), jnp.float32),
    )(v2d)


@jax.jit
def _run(k1d, v2d):
    ko = _sc_make(_NROWS)(k1d)
    vo = _tc_run(v2d)
    return ko, vo


def kernel(input_pos, k_val, v_val, k_packed, v_packed, k_mag, v_mag,
           k_mean, v_mean):
    shape = k_val.shape
    ko, vo = _run(k_val.reshape(-1), v_val.reshape(-1, _D))
    return ko.reshape(shape), vo.reshape(shape)


# speedup vs baseline: 1.1904x; 1.1238x over previous
import functools
import math

import jax
import jax.numpy as jnp
import numpy as np
from jax import lax
from jax.experimental import pallas as pl
from jax.experimental.pallas import tpu as pltpu
from jax.experimental.pallas import tpu_sc as plsc

_B, _H, _S, _D = 4, 16, 2048, 128
_NROWS = _B * _H * _S

_CENTROIDS = np.array(
    [-2.7326, -2.069, -1.618, -1.2562, -0.9423, -0.6568, -0.3881, -0.1284,
     0.1284, 0.3881, 0.6568, 0.9423, 1.2562, 1.618, 2.069, 2.7326],
    dtype=np.float32)
_BOUNDS = ((_CENTROIDS[:-1] + _CENTROIDS[1:]) / 2).astype(np.float32)
_PB = _BOUNDS[8:]
_C8 = float(_CENTROIDS[8])
_DCP = [float(x) for x in (_CENTROIDS[9:] - _CENTROIDS[8:15])]
_PB2D = [float(x) for x in (_PB.astype(np.float64) ** 2 / _D)]
_CPOS = [float(x) for x in _CENTROIDS[8:]]
_INV_SQRT_D = float(np.float32(1.0 / math.sqrt(_D)))

_NW = 32
_CHUNK = 128
_CS = _CHUNK * _D


def _rsqrt_newton(ssc):
    ii = lax.bitcast_convert_type(ssc, jnp.int32)
    ii = 0x5F3759DF - lax.shift_right_logical(ii, 1)
    y = lax.bitcast_convert_type(ii, jnp.float32)
    for _ in range(3):
        y = y * (1.5 - 0.5 * ssc * y * y)
    return y


def _sc_compute_chunk(inb_b, outb_b):
    def row_body(r, carry):
        base = r * _D
        xs = [inb_b[pl.ds(base + 16 * i, 16)] for i in range(8)]
        sm = ((xs[0] + xs[1]) + (xs[2] + xs[3])) + (
            (xs[4] + xs[5]) + (xs[6] + xs[7]))
        sq = [x * x for x in xs]
        sqm = ((sq[0] + sq[1]) + (sq[2] + sq[3])) + (
            (sq[4] + sq[5]) + (sq[6] + sq[7]))
        tot = jnp.sum(sm)
        tot2 = jnp.sum(sqm)
        mean = tot * (1.0 / _D)
        ssc = jnp.maximum(tot2 - mean * tot, 1e-30)
        mag = jnp.maximum(ssc * _rsqrt_newton(ssc), 1e-8)
        rm = mag * _INV_SQRT_D
        ones = jnp.full((16,), 1.0, jnp.float32)
        mean_v = ones * mean
        rm_v = ones * rm
        tbs = [ones * (pb2d * ssc) for pb2d in _PB2D]
        for i in range(8):
            xc = xs[i] - mean_v
            t = xc * xc
            acc = jnp.full((16,), _C8, jnp.float32)
            for j in range(7):
                acc = acc + jnp.where(t > tbs[j], _DCP[j], 0.0)
            val = jnp.where(xc > 0, acc, -acc) * rm_v + mean_v
            outb_b[pl.ds(base + 16 * i, 16)] = val
        return carry

    lax.fori_loop(0, _CHUNK, row_body, 0)


def _sc_make(n_rows, n_rows_total):
    rpw = n_rows // _NW
    nch = rpw // _CHUNK
    assert rpw % _CHUNK == 0 and nch % 2 == 0
    mesh = plsc.VectorSubcoreMesh(core_axis_name="c", subcore_axis_name="s")
    out = jax.ShapeDtypeStruct((n_rows_total * _D,), jnp.float32)

    @functools.partial(
        pl.kernel, mesh=mesh,
        out_type=out,
        compiler_params=pltpu.CompilerParams(needs_layout_passes=False),
        scratch_types=[
            pltpu.VMEM((_CS,), jnp.float32),
            pltpu.VMEM((_CS,), jnp.float32),
            pltpu.VMEM((_CS,), jnp.float32),
            pltpu.VMEM((_CS,), jnp.float32),
            pltpu.SemaphoreType.DMA,
            pltpu.SemaphoreType.DMA,
            pltpu.SemaphoreType.DMA,
            pltpu.SemaphoreType.DMA,
        ])
    def sc_kernel(src, dst, inb0, inb1, outb0, outb1, is0, is1, os0, os1):
        cid = lax.axis_index("c")
        sid = lax.axis_index("s")
        wid = sid * 2 + cid
        base = wid * (rpw * _D)
        inbs = (inb0, inb1)
        outbs = (outb0, outb1)
        isems = (is0, is1)
        osems = (os0, os1)

        pltpu.async_copy(src.at[pl.ds(base, _CS)], inbs[0], isems[0])

        def pair_body(p, carry):
            for b in (0, 1):
                i = 2 * p + b
                nb = 1 - b
                nxt = jnp.minimum(i + 1, nch - 1)
                pltpu.async_copy(
                    src.at[pl.ds(base + nxt * _CS, _CS)],
                    inbs[nb], isems[nb])
                pltpu.make_async_copy(
                    src.at[pl.ds(base + i * _CS, _CS)],
                    inbs[b], isems[b]).wait()
                @pl.when(i >= 2)
                def _():
                    pltpu.make_async_copy(
                        outbs[b],
                        dst.at[pl.ds(base + (i - 2) * _CS, _CS)],
                        osems[b]).wait()
                _sc_compute_chunk(inbs[b], outbs[b])
                pltpu.async_copy(
                    outbs[b],
                    dst.at[pl.ds(base + i * _CS, _CS)], osems[b])
            return carry

        lax.fori_loop(0, nch // 2, pair_body, 0)
        pltpu.make_async_copy(
            src.at[pl.ds(base, _CS)], inbs[0], isems[0]).wait()
        pltpu.make_async_copy(
            outbs[0],
            dst.at[pl.ds(base + (nch - 2) * _CS, _CS)], osems[0]).wait()
        pltpu.make_async_copy(
            outbs[1],
            dst.at[pl.ds(base + (nch - 1) * _CS, _CS)], osems[1]).wait()

    return sc_kernel


def _quant_dequant(x):
    mean = jnp.mean(x, axis=-1, keepdims=True)
    xc = x - mean
    ss = jnp.sum(xc * xc, axis=-1, keepdims=True)
    mag = jnp.maximum(jnp.sqrt(ss), 1e-8)
    rm = mag * _INV_SQRT_D
    a = jnp.abs(xc)
    acc = jnp.broadcast_to(_C8 * rm, x.shape)
    for j in range(7):
        acc = acc + jnp.where(a > float(_PB[j]) * rm, _DCP[j] * rm, 0.0)
    return jnp.where(xc > 0, acc, -acc) + mean


def _tc_body(v_ref, vo_ref):
    vo_ref[...] = _quant_dequant(v_ref[...])


def _tc_run(v2d):
    n = v2d.shape[0]
    blk = 2048
    spec = pl.BlockSpec((blk, _D), lambda i: (i, 0))
    return pl.pallas_call(
        _tc_body,
        grid=(n // blk,),
        in_specs=[spec],
        out_specs=spec,
        out_shape=jax.ShapeDtypeStruct((n, _D), jnp.float32),
    )(v2d)


def _tc_tail_body(src_ref, alias_ref, out_ref):
    del alias_ref
    out_ref[...] = _quant_dequant(src_ref[...])


def _tc_tail(k2d, sc_out2d, head_rows):
    n = k2d.shape[0]
    blk = 2048
    head_blocks = head_rows // blk
    spec = pl.BlockSpec((blk, _D), lambda i: (head_blocks + i, 0))
    return pl.pallas_call(
        _tc_tail_body,
        grid=((n - head_rows) // blk,),
        in_specs=[spec, pl.BlockSpec(memory_space=pltpu.MemorySpace.HBM)],
        out_specs=spec,
        out_shape=jax.ShapeDtypeStruct((n, _D), jnp.float32),
        input_output_aliases={1: 0},
    )(k2d, sc_out2d)


_R_SC = 90112


@jax.jit
def _run(k1d, k2d, v2d):
    sc_out = _sc_make(_R_SC, _NROWS)(k1d)
    vo = _tc_run(v2d)
    ko = _tc_tail(k2d, sc_out.reshape(_NROWS, _D), _R_SC)
    return ko, vo


def kernel(input_pos, k_val, v_val, k_packed, v_packed, k_mag, v_mag,
           k_mean, v_mean):
    shape = k_val.shape
    ko, vo = _run(k_val.reshape(-1), k_val.reshape(-1, _D),
                  v_val.reshape(-1, _D))
    return ko.reshape(shape), vo.reshape(shape)


# speedup vs baseline: 1.1961x; 1.0048x over previous
import functools
import math

import jax
import jax.numpy as jnp
import numpy as np
from jax import lax
from jax.experimental import pallas as pl
from jax.experimental.pallas import tpu as pltpu
from jax.experimental.pallas import tpu_sc as plsc

_B, _H, _S, _D = 4, 16, 2048, 128
_NROWS = _B * _H * _S

_CENTROIDS = np.array(
    [-2.7326, -2.069, -1.618, -1.2562, -0.9423, -0.6568, -0.3881, -0.1284,
     0.1284, 0.3881, 0.6568, 0.9423, 1.2562, 1.618, 2.069, 2.7326],
    dtype=np.float32)
_BOUNDS = ((_CENTROIDS[:-1] + _CENTROIDS[1:]) / 2).astype(np.float32)
_PB = _BOUNDS[8:]
_C8 = float(_CENTROIDS[8])
_DCP = [float(x) for x in (_CENTROIDS[9:] - _CENTROIDS[8:15])]
_PB2D = [float(x) for x in (_PB.astype(np.float64) ** 2 / _D)]
_CPOS = [float(x) for x in _CENTROIDS[8:]]
_INV_SQRT_D = float(np.float32(1.0 / math.sqrt(_D)))

_NW = 32
_CHUNK = 128
_CS = _CHUNK * _D


def _rsqrt_newton(ssc):
    ii = lax.bitcast_convert_type(ssc, jnp.int32)
    ii = 0x5F3759DF - lax.shift_right_logical(ii, 1)
    y = lax.bitcast_convert_type(ii, jnp.float32)
    for _ in range(3):
        y = y * (1.5 - 0.5 * ssc * y * y)
    return y


def _sc_compute_chunk(inb_b, outb_b):
    def row_body(r, carry):
        base = r * _D
        xs = [inb_b[pl.ds(base + 16 * i, 16)] for i in range(8)]
        sm = ((xs[0] + xs[1]) + (xs[2] + xs[3])) + (
            (xs[4] + xs[5]) + (xs[6] + xs[7]))
        sq = [x * x for x in xs]
        sqm = ((sq[0] + sq[1]) + (sq[2] + sq[3])) + (
            (sq[4] + sq[5]) + (sq[6] + sq[7]))
        tot = jnp.sum(sm)
        tot2 = jnp.sum(sqm)
        mean = tot * (1.0 / _D)
        ssc = jnp.maximum(tot2 - mean * tot, 1e-30)
        mag = jnp.maximum(ssc * _rsqrt_newton(ssc), 1e-8)
        rm = mag * _INV_SQRT_D
        ones = jnp.full((16,), 1.0, jnp.float32)
        mean_v = ones * mean
        rm_v = ones * rm
        tbs = [ones * (pb2d * ssc) for pb2d in _PB2D]
        for i in range(8):
            xc = xs[i] - mean_v
            t = xc * xc
            acc = jnp.full((16,), _C8, jnp.float32)
            for j in range(7):
                acc = acc + jnp.where(t > tbs[j], _DCP[j], 0.0)
            val = jnp.where(xc > 0, acc, -acc) * rm_v + mean_v
            outb_b[pl.ds(base + 16 * i, 16)] = val
        return carry

    lax.fori_loop(0, _CHUNK, row_body, 0)


def _sc_make(n_rows, n_rows_total):
    rpw = n_rows // _NW
    nch = rpw // _CHUNK
    assert rpw % _CHUNK == 0 and nch % 2 == 0
    mesh = plsc.VectorSubcoreMesh(core_axis_name="c", subcore_axis_name="s")
    out = jax.ShapeDtypeStruct((n_rows_total * _D,), jnp.float32)

    @functools.partial(
        pl.kernel, mesh=mesh,
        out_type=out,
        compiler_params=pltpu.CompilerParams(needs_layout_passes=False),
        scratch_types=[
            pltpu.VMEM((_CS,), jnp.float32),
            pltpu.VMEM((_CS,), jnp.float32),
            pltpu.VMEM((_CS,), jnp.float32),
            pltpu.VMEM((_CS,), jnp.float32),
            pltpu.SemaphoreType.DMA,
            pltpu.SemaphoreType.DMA,
            pltpu.SemaphoreType.DMA,
            pltpu.SemaphoreType.DMA,
        ])
    def sc_kernel(src, dst, inb0, inb1, outb0, outb1, is0, is1, os0, os1):
        cid = lax.axis_index("c")
        sid = lax.axis_index("s")
        wid = sid * 2 + cid
        base = wid * (rpw * _D)
        inbs = (inb0, inb1)
        outbs = (outb0, outb1)
        isems = (is0, is1)
        osems = (os0, os1)

        pltpu.async_copy(src.at[pl.ds(base, _CS)], inbs[0], isems[0])

        def pair_body(p, carry):
            for b in (0, 1):
                i = 2 * p + b
                nb = 1 - b
                nxt = jnp.minimum(i + 1, nch - 1)
                pltpu.async_copy(
                    src.at[pl.ds(base + nxt * _CS, _CS)],
                    inbs[nb], isems[nb])
                pltpu.make_async_copy(
                    src.at[pl.ds(base + i * _CS, _CS)],
                    inbs[b], isems[b]).wait()
                @pl.when(i >= 2)
                def _():
                    pltpu.make_async_copy(
                        outbs[b],
                        dst.at[pl.ds(base + (i - 2) * _CS, _CS)],
                        osems[b]).wait()
                _sc_compute_chunk(inbs[b], outbs[b])
                pltpu.async_copy(
                    outbs[b],
                    dst.at[pl.ds(base + i * _CS, _CS)], osems[b])
            return carry

        lax.fori_loop(0, nch // 2, pair_body, 0)
        pltpu.make_async_copy(
            src.at[pl.ds(base, _CS)], inbs[0], isems[0]).wait()
        pltpu.make_async_copy(
            outbs[0],
            dst.at[pl.ds(base + (nch - 2) * _CS, _CS)], osems[0]).wait()
        pltpu.make_async_copy(
            outbs[1],
            dst.at[pl.ds(base + (nch - 1) * _CS, _CS)], osems[1]).wait()

    return sc_kernel


def _quant_dequant(x):
    mean = jnp.mean(x, axis=-1, keepdims=True)
    xc = x - mean
    ss = jnp.sum(xc * xc, axis=-1, keepdims=True)
    mag = jnp.maximum(jnp.sqrt(ss), 1e-8)
    rm = mag * _INV_SQRT_D
    a = jnp.abs(xc)
    acc = jnp.broadcast_to(_C8 * rm, x.shape)
    for j in range(7):
        acc = acc + jnp.where(a > float(_PB[j]) * rm, _DCP[j] * rm, 0.0)
    return jnp.where(xc > 0, acc, -acc) + mean


def _tc_body(v_ref, vo_ref):
    vo_ref[...] = _quant_dequant(v_ref[...])


def _tc_run(v2d):
    n = v2d.shape[0]
    blk = 4096
    spec = pl.BlockSpec((blk, _D), lambda i: (i, 0))
    return pl.pallas_call(
        _tc_body,
        grid=(n // blk,),
        in_specs=[spec],
        out_specs=spec,
        out_shape=jax.ShapeDtypeStruct((n, _D), jnp.float32),
    )(v2d)


def _tc_tail_body(src_ref, alias_ref, out_ref):
    del alias_ref
    out_ref[...] = _quant_dequant(src_ref[...])


def _tc_tail(k2d, sc_out2d, head_rows):
    n = k2d.shape[0]
    blk = 4096
    head_blocks = head_rows // blk
    spec = pl.BlockSpec((blk, _D), lambda i: (head_blocks + i, 0))
    return pl.pallas_call(
        _tc_tail_body,
        grid=((n - head_rows) // blk,),
        in_specs=[spec, pl.BlockSpec(memory_space=pltpu.MemorySpace.HBM)],
        out_specs=spec,
        out_shape=jax.ShapeDtypeStruct((n, _D), jnp.float32),
        input_output_aliases={1: 0},
    )(k2d, sc_out2d)


_R_SC = 90112


@jax.jit
def _run(k1d, k2d, v2d):
    sc_out = _sc_make(_R_SC, _NROWS)(k1d)
    vo = _tc_run(v2d)
    ko = _tc_tail(k2d, sc_out.reshape(_NROWS, _D), _R_SC)
    return ko, vo


def kernel(input_pos, k_val, v_val, k_packed, v_packed, k_mag, v_mag,
           k_mean, v_mean):
    shape = k_val.shape
    ko, vo = _run(k_val.reshape(-1), k_val.reshape(-1, _D),
                  v_val.reshape(-1, _D))
    return ko.reshape(shape), vo.reshape(shape)


# speedup vs baseline: 1.1984x; 1.0020x over previous
import functools
import math

import jax
import jax.numpy as jnp
import numpy as np
from jax import lax
from jax.experimental import pallas as pl
from jax.experimental.pallas import tpu as pltpu
from jax.experimental.pallas import tpu_sc as plsc

_B, _H, _S, _D = 4, 16, 2048, 128
_NROWS = _B * _H * _S

_CENTROIDS = np.array(
    [-2.7326, -2.069, -1.618, -1.2562, -0.9423, -0.6568, -0.3881, -0.1284,
     0.1284, 0.3881, 0.6568, 0.9423, 1.2562, 1.618, 2.069, 2.7326],
    dtype=np.float32)
_BOUNDS = ((_CENTROIDS[:-1] + _CENTROIDS[1:]) / 2).astype(np.float32)
_PB = _BOUNDS[8:]
_C8 = float(_CENTROIDS[8])
_DCP = [float(x) for x in (_CENTROIDS[9:] - _CENTROIDS[8:15])]
_PB2D = [float(x) for x in (_PB.astype(np.float64) ** 2 / _D)]
_INV_SQRT_D = float(np.float32(1.0 / math.sqrt(_D)))

_NW = 32
_CHUNK = 128
_CS = _CHUNK * _D


def _rsqrt_newton(ssc):
    ii = lax.bitcast_convert_type(ssc, jnp.int32)
    ii = 0x5F3759DF - lax.shift_right_logical(ii, 1)
    y = lax.bitcast_convert_type(ii, jnp.float32)
    for _ in range(3):
        y = y * (1.5 - 0.5 * ssc * y * y)
    return y


def _sc_compute_chunk(inb_b, outb_b):
    def row_body(r, carry):
        base = r * _D
        xs = [inb_b[pl.ds(base + 16 * i, 16)] for i in range(8)]
        sm = ((xs[0] + xs[1]) + (xs[2] + xs[3])) + (
            (xs[4] + xs[5]) + (xs[6] + xs[7]))
        sq = [x * x for x in xs]
        sqm = ((sq[0] + sq[1]) + (sq[2] + sq[3])) + (
            (sq[4] + sq[5]) + (sq[6] + sq[7]))
        tot = jnp.sum(sm)
        tot2 = jnp.sum(sqm)
        mean = tot * (1.0 / _D)
        ssc = jnp.maximum(tot2 - mean * tot, 1e-30)
        mag = jnp.maximum(ssc * _rsqrt_newton(ssc), 1e-8)
        rm = mag * _INV_SQRT_D
        ones = jnp.full((16,), 1.0, jnp.float32)
        mean_v = ones * mean
        rm_v = ones * rm
        tbs = [ones * (pb2d * ssc) for pb2d in _PB2D]
        for i in range(8):
            xc = xs[i] - mean_v
            t = xc * xc
            acc = jnp.full((16,), _C8, jnp.float32)
            for j in range(7):
                acc = acc + jnp.where(t > tbs[j], _DCP[j], 0.0)
            val = jnp.where(xc > 0, acc, -acc) * rm_v + mean_v
            outb_b[pl.ds(base + 16 * i, 16)] = val
        return carry

    lax.fori_loop(0, _CHUNK, row_body, 0)


def _sc_make(n_rows, n_rows_total):
    rpw = n_rows // _NW
    nch = rpw // _CHUNK
    assert rpw % _CHUNK == 0 and nch % 2 == 0
    mesh = plsc.VectorSubcoreMesh(core_axis_name="c", subcore_axis_name="s")
    out = jax.ShapeDtypeStruct((n_rows_total * _D,), jnp.float32)

    @functools.partial(
        pl.kernel, mesh=mesh,
        out_type=out,
        compiler_params=pltpu.CompilerParams(needs_layout_passes=False),
        scratch_types=[
            pltpu.VMEM((_CS,), jnp.float32),
            pltpu.VMEM((_CS,), jnp.float32),
            pltpu.VMEM((_CS,), jnp.float32),
            pltpu.VMEM((_CS,), jnp.float32),
            pltpu.SemaphoreType.DMA,
            pltpu.SemaphoreType.DMA,
            pltpu.SemaphoreType.DMA,
            pltpu.SemaphoreType.DMA,
        ])
    def sc_kernel(src, dst, inb0, inb1, outb0, outb1, is0, is1, os0, os1):
        cid = lax.axis_index("c")
        sid = lax.axis_index("s")
        wid = sid * 2 + cid
        base = wid * (rpw * _D)
        inbs = (inb0, inb1)
        outbs = (outb0, outb1)
        isems = (is0, is1)
        osems = (os0, os1)

        pltpu.async_copy(src.at[pl.ds(base, _CS)], inbs[0], isems[0])

        def pair_body(p, carry):
            for b in (0, 1):
                i = 2 * p + b
                nb = 1 - b
                nxt = jnp.minimum(i + 1, nch - 1)
                pltpu.async_copy(
                    src.at[pl.ds(base + nxt * _CS, _CS)],
                    inbs[nb], isems[nb])
                pltpu.make_async_copy(
                    src.at[pl.ds(base + i * _CS, _CS)],
                    inbs[b], isems[b]).wait()
                @pl.when(i >= 2)
                def _():
                    pltpu.make_async_copy(
                        outbs[b],
                        dst.at[pl.ds(base + (i - 2) * _CS, _CS)],
                        osems[b]).wait()
                _sc_compute_chunk(inbs[b], outbs[b])
                pltpu.async_copy(
                    outbs[b],
                    dst.at[pl.ds(base + i * _CS, _CS)], osems[b])
            return carry

        lax.fori_loop(0, nch // 2, pair_body, 0)
        pltpu.make_async_copy(
            src.at[pl.ds(base, _CS)], inbs[0], isems[0]).wait()
        pltpu.make_async_copy(
            outbs[0],
            dst.at[pl.ds(base + (nch - 2) * _CS, _CS)], osems[0]).wait()
        pltpu.make_async_copy(
            outbs[1],
            dst.at[pl.ds(base + (nch - 1) * _CS, _CS)], osems[1]).wait()

    return sc_kernel


def _quant_dequant(x):
    mean = jnp.mean(x, axis=-1, keepdims=True)
    xc = x - mean
    ss = jnp.sum(xc * xc, axis=-1, keepdims=True)
    mag = jnp.maximum(jnp.sqrt(ss), 1e-8)
    rm = mag * _INV_SQRT_D
    a = jnp.abs(xc)
    acc = jnp.broadcast_to(_C8 * rm, x.shape)
    for j in range(7):
        acc = acc + jnp.where(a > float(_PB[j]) * rm, _DCP[j] * rm, 0.0)
    return jnp.where(xc > 0, acc, -acc) + mean


def _tc_body(v_ref, vo_ref):
    vo_ref[...] = _quant_dequant(v_ref[...])


def _tc_run(v2d):
    n = v2d.shape[0]
    blk = 4096
    spec = pl.BlockSpec((blk, _D), lambda i: (i, 0))
    return pl.pallas_call(
        _tc_body,
        grid=(n // blk,),
        in_specs=[spec],
        out_specs=spec,
        out_shape=jax.ShapeDtypeStruct((n, _D), jnp.float32),
    )(v2d)


def _tc_tail_body(src_ref, alias_ref, out_ref):
    del alias_ref
    out_ref[...] = _quant_dequant(src_ref[...])


def _tc_tail(k2d, sc_out2d, head_rows):
    n = k2d.shape[0]
    blk = 4096
    head_blocks = head_rows // blk
    spec = pl.BlockSpec((blk, _D), lambda i: (head_blocks + i, 0))
    return pl.pallas_call(
        _tc_tail_body,
        grid=((n - head_rows) // blk,),
        in_specs=[spec, pl.BlockSpec(memory_space=pltpu.MemorySpace.HBM)],
        out_specs=spec,
        out_shape=jax.ShapeDtypeStruct((n, _D), jnp.float32),
        input_output_aliases={1: 0},
    )(k2d, sc_out2d)


_R_SC = 90112


@jax.jit
def _run(k1d, k2d, v2d):
    sc_out = _sc_make(_R_SC, _NROWS)(k1d)
    vo = _tc_run(v2d)
    ko = _tc_tail(k2d, sc_out.reshape(_NROWS, _D), _R_SC)
    return ko, vo


def kernel(input_pos, k_val, v_val, k_packed, v_packed, k_mag, v_mag,
           k_mean, v_mean):
    shape = k_val.shape
    ko, vo = _run(k_val.reshape(-1), k_val.reshape(-1, _D),
                  v_val.reshape(-1, _D))
    return ko.reshape(shape), vo.reshape(shape)
